# baseline (speedup 1.0000x reference)
import functools

import jax
import jax.numpy as jnp
from jax import lax
from jax.experimental import pallas as pl
from jax.experimental.pallas import tpu as pltpu
from jax.experimental.pallas import tpu_sc as plsc

NUM_FIELDS = 26
NUM_EMBEDDINGS = 100001
EMBED_DIM = 32
BATCH = 4096

NUM_CORES = 2
NUM_SUBCORES = 16
LANES = 16
NUM_WORKERS = NUM_CORES * NUM_SUBCORES

VT = -(-NUM_EMBEDDINGS // 128)
VTF = NUM_EMBEDDINGS // 128
NTILES = NUM_FIELDS * 4 * VT
NTC = NUM_FIELDS * 4 * VTF
TPW = -(-NTC // NUM_WORKERS)
NTAIL = NUM_FIELDS * 4
FROW = VT * 1024

BPW = BATCH // NUM_WORKERS
QB = 32
NQ = BPW // QB
CHUNK = 128
NCH = NUM_FIELDS * EMBED_DIM * QB // CHUNK


def _copy_body(tab_hbm, tail_hbm, out_hbm, stg0, stg1, sem0, sem1, wsem):
    wid = lax.axis_index("s") * NUM_CORES + lax.axis_index("c")

    stgs = (stg0, stg1)
    sems = (sem0, sem1)

    def parts(t):
        f = t // (4 * VTF)
        r = lax.rem(t, 4 * VTF)
        dt = r // VTF
        c = lax.rem(r, VTF)
        return f, dt, c

    def src(t):
        f, dt, c = parts(t)
        d8 = pl.multiple_of(dt * 8, 8)
        v0 = pl.multiple_of(c * 128, 128)
        return tab_hbm.at[f, pl.ds(d8, 8), pl.ds(v0, 128)]

    def dst_idx(t):
        f, dt, c = parts(t)
        return (f * 4 + dt) * VT + c

    W = 32

    def step(j, carry):
        t = wid + j * NUM_WORKERS

        @pl.when(t < NTC)
        def _fire():
            pltpu.async_copy(src(t), out_hbm.at[dst_idx(t)], wsem)

        @pl.when((j >= W) & (t < NTC + W * NUM_WORKERS))
        def _lag():
            pltpu.make_async_copy(src(wid), out_hbm.at[0], wsem).wait()

        return carry

    lax.fori_loop(0, TPW + W, step, 0)

    for k in range(-(-NTAIL // NUM_WORKERS)):
        u = wid + k * NUM_WORKERS

        @pl.when(u < NTAIL)
        def _tail(u=u):
            pltpu.sync_copy(tail_hbm.at[u], out_hbm.at[u * VT + VTF])


def _gather_body(cat_hbm, tab_hbm, out_hbm, catv, idxv, dstv, sem):
    wid = lax.axis_index("s") * NUM_CORES + lax.axis_index("c")
    b0 = wid * BPW

    pltpu.sync_copy(cat_hbm.at[:, pl.ds(b0, BPW)], catv)

    for q in range(NQ):
        for f in range(NUM_FIELDS):
            vv0 = catv[f, pl.ds(q * QB, LANES)]
            vv1 = catv[f, pl.ds(q * QB + LANES, LANES)]
            vt0 = ((vv0 >> 7) << 10) + (vv0 & 127)
            vt1 = ((vv1 >> 7) << 10) + (vv1 & 127)

            def build(d, carry, vt0=vt0, vt1=vt1, f=f):
                base = (f * 4 + (d >> 3)) * FROW + (d & 7) * CHUNK
                j0 = (f * EMBED_DIM + d) * QB
                row = j0 >> 7
                col = j0 & (CHUNK - 1)
                idxv[row, pl.ds(col, LANES)] = vt0 + base
                idxv[row, pl.ds(col + LANES, LANES)] = vt1 + base
                return carry

            lax.fori_loop(0, EMBED_DIM, build, 0)

        def fire(c, carry):
            pltpu.async_copy(tab_hbm.at[idxv.at[c]], dstv.at[c], sem)
            return carry

        lax.fori_loop(0, NCH, fire, 0)
        pltpu.make_async_copy(out_hbm.at[wid, q], dstv, sem).wait()

        pltpu.sync_copy(dstv, out_hbm.at[wid, q])


@jax.jit
def _cat_embedding(cat_features, tables_t, tail_tiles):
    mesh = plsc.VectorSubcoreMesh(core_axis_name="c", subcore_axis_name="s")

    copy_run = pl.kernel(
        _copy_body,
        out_type=jax.ShapeDtypeStruct((NTILES, 8, 128), jnp.float32),
        mesh=mesh,
        scratch_types=[
            pltpu.VMEM((8, 128), jnp.float32),
            pltpu.VMEM((8, 128), jnp.float32),
            pltpu.SemaphoreType.DMA,
            pltpu.SemaphoreType.DMA,
            pltpu.SemaphoreType.DMA,
        ],
        compiler_params=pltpu.CompilerParams(
            needs_layout_passes=False, use_tc_tiling_on_sc=True
        ),
    )
    flat = copy_run(tables_t, tail_tiles).reshape(NTILES * 1024)

    gather_run = pl.kernel(
        _gather_body,
        out_type=jax.ShapeDtypeStruct(
            (NUM_WORKERS, NQ, NCH, CHUNK), jnp.float32
        ),
        mesh=mesh,
        scratch_types=[
            pltpu.VMEM((NUM_FIELDS, BPW), jnp.int32),
            pltpu.VMEM((NCH, CHUNK), jnp.int32),
            pltpu.VMEM((NCH, CHUNK), jnp.float32),
            pltpu.SemaphoreType.DMA,
        ],
        compiler_params=pltpu.CompilerParams(
            needs_layout_passes=False, use_tc_tiling_on_sc=False
        ),
    )
    return gather_run(cat_features, flat)


def kernel(cat_features, tables):
    cat = cat_features.astype(jnp.int32)
    tab_t = jnp.transpose(tables, (0, 2, 1))
    tail = jnp.transpose(tables[:, VTF * 128 :, :], (0, 2, 1))
    tail = jnp.pad(tail, ((0, 0), (0, 0), (0, 128 - (NUM_EMBEDDINGS - VTF * 128))))
    out = _cat_embedding(cat, tab_t, tail.reshape(NTAIL, 8, 128))
    out = out.reshape(NUM_WORKERS, NQ, NUM_FIELDS, EMBED_DIM, QB)
    out = jnp.transpose(out, (0, 1, 4, 2, 3))
    return out.reshape(BATCH, NUM_FIELDS * EMBED_DIM)


# speedup vs baseline: 5.6439x; 5.6439x over previous
import functools

import jax
import jax.numpy as jnp
from jax import lax
from jax.experimental import pallas as pl
from jax.experimental.pallas import tpu as pltpu
from jax.experimental.pallas import tpu_sc as plsc

NUM_FIELDS = 26
NUM_EMBEDDINGS = 100001
EMBED_DIM = 32
BATCH = 4096

NUM_CORES = 2
NUM_SUBCORES = 16
LANES = 16
NUM_WORKERS = NUM_CORES * NUM_SUBCORES

VT = -(-NUM_EMBEDDINGS // 128)
VTF = NUM_EMBEDDINGS // 128
NTILES = NUM_FIELDS * 4 * VT
NTC = NUM_FIELDS * 4 * VTF
TPW = -(-NTC // NUM_WORKERS)
NTAIL = NUM_FIELDS * 4
FROW = VT * 1024

BPW = BATCH // NUM_WORKERS
QB = 32
NQ = BPW // QB
CHUNK = 128
NCH = NUM_FIELDS * EMBED_DIM * QB // CHUNK


def _copy_body(tab_hbm, tail_hbm, out_hbm, stg0, stg1, sem0, sem1, wsem):
    wid = lax.axis_index("s") * NUM_CORES + lax.axis_index("c")

    stgs = (stg0, stg1)
    sems = (sem0, sem1)

    def parts(t):
        f = t // (4 * VTF)
        r = lax.rem(t, 4 * VTF)
        dt = r // VTF
        c = lax.rem(r, VTF)
        return f, dt, c

    def src(t):
        f, dt, c = parts(t)
        d8 = pl.multiple_of(dt * 8, 8)
        v0 = pl.multiple_of(c * 128, 128)
        return tab_hbm.at[f, pl.ds(d8, 8), pl.ds(v0, 128)]

    def dst_idx(t):
        f, dt, c = parts(t)
        return (f * 4 + dt) * VT + c

    pltpu.async_copy(src(wid), stg0, sem0)

    def step(j, carry):
        t = wid + j * NUM_WORKERS
        for p in range(2):
            @pl.when((lax.rem(j, 2) == p) & (t < NTC))
            def _go(p=p, t=t, j=j):
                stg, nstg = stgs[p], stgs[1 - p]
                sem, nsem = sems[p], sems[1 - p]
                pltpu.make_async_copy(src(t), stg, sem).wait()
                tn = t + NUM_WORKERS

                @pl.when(tn < NTC)
                def _pref():
                    pltpu.async_copy(src(tn), nstg, nsem)

                pltpu.async_copy(stg, out_hbm.at[dst_idx(t)], wsem)

                @pl.when(j >= 2)
                def _lag():
                    pltpu.make_async_copy(out_hbm.at[0], stg, wsem).wait()

        return carry

    lax.fori_loop(0, TPW, step, 0)
    for _ in range(2):
        pltpu.make_async_copy(out_hbm.at[0], stg0, wsem).wait()

    for k in range(-(-NTAIL // NUM_WORKERS)):
        u = wid + k * NUM_WORKERS

        @pl.when(u < NTAIL)
        def _tail(u=u):
            pltpu.sync_copy(tail_hbm.at[u], out_hbm.at[u * VT + VTF])


def _gather_body(cat_hbm, tab_hbm, out_hbm, catv, idxv, dstv, sem):
    wid = lax.axis_index("s") * NUM_CORES + lax.axis_index("c")
    b0 = wid * BPW

    pltpu.sync_copy(cat_hbm.at[:, pl.ds(b0, BPW)], catv)

    for q in range(NQ):
        for f in range(NUM_FIELDS):
            vv0 = catv[f, pl.ds(q * QB, LANES)]
            vv1 = catv[f, pl.ds(q * QB + LANES, LANES)]
            vt0 = ((vv0 >> 7) << 10) + (vv0 & 127)
            vt1 = ((vv1 >> 7) << 10) + (vv1 & 127)

            def build(d, carry, vt0=vt0, vt1=vt1, f=f):
                base = (f * 4 + (d >> 3)) * FROW + (d & 7) * CHUNK
                j0 = (f * EMBED_DIM + d) * QB
                row = j0 >> 7
                col = j0 & (CHUNK - 1)
                idxv[row, pl.ds(col, LANES)] = vt0 + base
                idxv[row, pl.ds(col + LANES, LANES)] = vt1 + base
                return carry

            lax.fori_loop(0, EMBED_DIM, build, 0)

        def fire(c, carry):
            pltpu.async_copy(tab_hbm.at[idxv.at[c]], dstv.at[c], sem)
            return carry

        lax.fori_loop(0, NCH, fire, 0)
        pltpu.make_async_copy(out_hbm.at[wid, q], dstv, sem).wait()

        pltpu.sync_copy(dstv, out_hbm.at[wid, q])


@jax.jit
def _cat_embedding(cat_features, tables_t, tail_tiles):
    mesh = plsc.VectorSubcoreMesh(core_axis_name="c", subcore_axis_name="s")

    copy_run = pl.kernel(
        _copy_body,
        out_type=jax.ShapeDtypeStruct((NTILES, 8, 128), jnp.float32),
        mesh=mesh,
        scratch_types=[
            pltpu.VMEM((8, 128), jnp.float32),
            pltpu.VMEM((8, 128), jnp.float32),
            pltpu.SemaphoreType.DMA,
            pltpu.SemaphoreType.DMA,
            pltpu.SemaphoreType.DMA,
        ],
        compiler_params=pltpu.CompilerParams(
            needs_layout_passes=False, use_tc_tiling_on_sc=True
        ),
    )
    flat = copy_run(tables_t, tail_tiles).reshape(NTILES * 1024)

    gather_run = pl.kernel(
        _gather_body,
        out_type=jax.ShapeDtypeStruct(
            (NUM_WORKERS, NQ, NCH, CHUNK), jnp.float32
        ),
        mesh=mesh,
        scratch_types=[
            pltpu.VMEM((NUM_FIELDS, BPW), jnp.int32),
            pltpu.VMEM((NCH, CHUNK), jnp.int32),
            pltpu.VMEM((NCH, CHUNK), jnp.float32),
            pltpu.SemaphoreType.DMA,
        ],
        compiler_params=pltpu.CompilerParams(
            needs_layout_passes=False, use_tc_tiling_on_sc=False
        ),
    )
    return gather_run(cat_features, flat)


def kernel(cat_features, tables):
    cat = cat_features.astype(jnp.int32)
    tab_t = jnp.transpose(tables, (0, 2, 1))
    tail = jnp.transpose(tables[:, 99968:, :], (0, 2, 1))
    tail = jnp.pad(tail, ((0, 0), (0, 0), (0, 95)))
    out = _cat_embedding(cat, tab_t, tail.reshape(NTAIL, 8, 128))
    out = out.reshape(NUM_WORKERS, NQ, NUM_FIELDS, EMBED_DIM, QB)
    out = jnp.transpose(out, (0, 1, 4, 2, 3))
    return out.reshape(BATCH, NUM_FIELDS * EMBED_DIM)


# speedup vs baseline: 14.7223x; 2.6085x over previous
import functools

import jax
import jax.numpy as jnp
from jax import lax
from jax.experimental import pallas as pl
from jax.experimental.pallas import tpu as pltpu
from jax.experimental.pallas import tpu_sc as plsc

NUM_FIELDS = 26
NUM_EMBEDDINGS = 100001
EMBED_DIM = 32
BATCH = 4096

NUM_CORES = 2
NUM_SUBCORES = 16
LANES = 16
NUM_WORKERS = NUM_CORES * NUM_SUBCORES

VT = -(-NUM_EMBEDDINGS // 128)
VTF = NUM_EMBEDDINGS // 128
NTILES = NUM_FIELDS * 4 * VT
NTC = NUM_FIELDS * 4 * VTF
TPW = -(-NTC // NUM_WORKERS)
NTAIL = NUM_FIELDS * 4
FROW = VT * 1024

BPW = BATCH // NUM_WORKERS
QB = 32
NQ = BPW // QB
CHUNK = 128
NCH = NUM_FIELDS * EMBED_DIM * QB // CHUNK


def _copy_body(tab_hbm, tail_hbm, out_hbm, *scr):
    stgs = scr[:6]
    sems = scr[6:12]
    wsem = scr[12]
    wid = lax.axis_index("s") * NUM_CORES + lax.axis_index("c")

    def parts(t):
        f = t // (4 * VTF)
        r = lax.rem(t, 4 * VTF)
        dt = r // VTF
        c = lax.rem(r, VTF)
        return f, dt, c

    def src(t):
        f, dt, c = parts(t)
        d8 = pl.multiple_of(dt * 8, 8)
        v0 = pl.multiple_of(c * 128, 128)
        return tab_hbm.at[f, pl.ds(d8, 8), pl.ds(v0, 128)]

    def dst_idx(t):
        f, dt, c = parts(t)
        return (f * 4 + dt) * VT + c

    K = 6
    for k in range(K - 1):
        pltpu.async_copy(src(wid + k * NUM_WORKERS), stgs[k], sems[k])

    def step(j, carry):
        t = wid + j * NUM_WORKERS
        for p in range(K):
            @pl.when((lax.rem(j, K) == p) & (t < NTC))
            def _go(p=p, t=t, j=j):
                stg, sem = stgs[p], sems[p]
                pltpu.make_async_copy(src(t), stg, sem).wait()
                tn = t + (K - 1) * NUM_WORKERS
                q = (p + K - 1) % K

                @pl.when(tn < NTC)
                def _pref():
                    pltpu.async_copy(src(tn), stgs[q], sems[q])

                pltpu.async_copy(stg, out_hbm.at[dst_idx(t)], wsem)

                @pl.when(j >= 2)
                def _lag():
                    pltpu.make_async_copy(out_hbm.at[0], stg, wsem).wait()

        return carry

    lax.fori_loop(0, TPW, step, 0)
    for _ in range(2):
        pltpu.make_async_copy(out_hbm.at[0], stgs[0], wsem).wait()

    for k in range(-(-NTAIL // NUM_WORKERS)):
        u = wid + k * NUM_WORKERS

        @pl.when(u < NTAIL)
        def _tail(u=u):
            pltpu.sync_copy(tail_hbm.at[u], out_hbm.at[u * VT + VTF])


def _gather_body(cat_hbm, tab_hbm, out_hbm, catv, idxv, dstv, sem):
    wid = lax.axis_index("s") * NUM_CORES + lax.axis_index("c")
    b0 = wid * BPW

    pltpu.sync_copy(cat_hbm.at[:, pl.ds(b0, BPW)], catv)

    for q in range(NQ):
        for f in range(NUM_FIELDS):
            vv0 = catv[f, pl.ds(q * QB, LANES)]
            vv1 = catv[f, pl.ds(q * QB + LANES, LANES)]
            vt0 = ((vv0 >> 7) << 10) + (vv0 & 127)
            vt1 = ((vv1 >> 7) << 10) + (vv1 & 127)

            def build(d, carry, vt0=vt0, vt1=vt1, f=f):
                base = (f * 4 + (d >> 3)) * FROW + (d & 7) * CHUNK
                j0 = (f * EMBED_DIM + d) * QB
                row = j0 >> 7
                col = j0 & (CHUNK - 1)
                idxv[row, pl.ds(col, LANES)] = vt0 + base
                idxv[row, pl.ds(col + LANES, LANES)] = vt1 + base
                return carry

            lax.fori_loop(0, EMBED_DIM, build, 0)

        def fire(c, carry):
            pltpu.async_copy(tab_hbm.at[idxv.at[c]], dstv.at[c], sem)
            return carry

        lax.fori_loop(0, NCH, fire, 0)
        pltpu.make_async_copy(out_hbm.at[wid, q], dstv, sem).wait()

        pltpu.sync_copy(dstv, out_hbm.at[wid, q])


@jax.jit
def _cat_embedding(cat_features, tables_t, tail_tiles):
    mesh = plsc.VectorSubcoreMesh(core_axis_name="c", subcore_axis_name="s")

    copy_run = pl.kernel(
        _copy_body,
        out_type=jax.ShapeDtypeStruct((NTILES, 8, 128), jnp.float32),
        mesh=mesh,
        scratch_types=(
            [pltpu.VMEM((8, 128), jnp.float32)] * 6
            + [pltpu.SemaphoreType.DMA] * 7
        ),
        compiler_params=pltpu.CompilerParams(
            needs_layout_passes=False, use_tc_tiling_on_sc=True
        ),
    )
    flat = copy_run(tables_t, tail_tiles).reshape(NTILES * 1024)

    gather_run = pl.kernel(
        _gather_body,
        out_type=jax.ShapeDtypeStruct(
            (NUM_WORKERS, NQ, NCH, CHUNK), jnp.float32
        ),
        mesh=mesh,
        scratch_types=[
            pltpu.VMEM((NUM_FIELDS, BPW), jnp.int32),
            pltpu.VMEM((NCH, CHUNK), jnp.int32),
            pltpu.VMEM((NCH, CHUNK), jnp.float32),
            pltpu.SemaphoreType.DMA,
        ],
        compiler_params=pltpu.CompilerParams(
            needs_layout_passes=False, use_tc_tiling_on_sc=False
        ),
    )
    return gather_run(cat_features, flat)


def kernel(cat_features, tables):
    cat = cat_features.astype(jnp.int32)
    tab_t = jnp.transpose(tables, (0, 2, 1))
    tail = jnp.transpose(tables[:, 99968:, :], (0, 2, 1))
    tail = jnp.pad(tail, ((0, 0), (0, 0), (0, 95)))
    out = _cat_embedding(cat, tab_t, tail.reshape(NTAIL, 8, 128))
    out = out.reshape(NUM_WORKERS, NQ, NUM_FIELDS, EMBED_DIM, QB)
    out = jnp.transpose(out, (0, 1, 4, 2, 3))
    return out.reshape(BATCH, NUM_FIELDS * EMBED_DIM)


# speedup vs baseline: 17.8931x; 1.2154x over previous
import functools

import jax
import jax.numpy as jnp
from jax import lax
from jax.experimental import pallas as pl
from jax.experimental.pallas import tpu as pltpu
from jax.experimental.pallas import tpu_sc as plsc

NUM_FIELDS = 26
NUM_EMBEDDINGS = 100001
EMBED_DIM = 32
BATCH = 4096

NUM_CORES = 2
NUM_SUBCORES = 16
LANES = 16
NUM_WORKERS = NUM_CORES * NUM_SUBCORES

VT = -(-NUM_EMBEDDINGS // 128)
VTF = NUM_EMBEDDINGS // 128
NTILES = NUM_FIELDS * 4 * VT
NTC = NUM_FIELDS * 4 * VTF
TPW = -(-NTC // NUM_WORKERS)
NTAIL = NUM_FIELDS * 4
FROW = VT * 1024

BPW = BATCH // NUM_WORKERS
QB = 32
NQ = BPW // QB
CHUNK = 128
NCH = NUM_FIELDS * EMBED_DIM * QB // CHUNK


def _copy_body(tab_hbm, tail_hbm, out_hbm, *scr):
    stgs = scr[:12]
    sems = scr[12:24]
    wsem = scr[24]
    wid = lax.axis_index("s") * NUM_CORES + lax.axis_index("c")

    def parts(t):
        f = t // (4 * VTF)
        r = lax.rem(t, 4 * VTF)
        dt = r // VTF
        c = lax.rem(r, VTF)
        return f, dt, c

    def src(t):
        f, dt, c = parts(t)
        d8 = pl.multiple_of(dt * 8, 8)
        v0 = pl.multiple_of(c * 128, 128)
        return tab_hbm.at[f, pl.ds(d8, 8), pl.ds(v0, 128)]

    def dst_idx(t):
        f, dt, c = parts(t)
        return (f * 4 + dt) * VT + c

    K = 12
    for k in range(K - 1):
        pltpu.async_copy(src(wid + k * NUM_WORKERS), stgs[k], sems[k])

    def step(j, carry):
        t = wid + j * NUM_WORKERS
        for p in range(K):
            @pl.when((lax.rem(j, K) == p) & (t < NTC))
            def _go(p=p, t=t, j=j):
                stg, sem = stgs[p], sems[p]
                pltpu.make_async_copy(src(t), stg, sem).wait()
                tn = t + (K - 1) * NUM_WORKERS
                q = (p + K - 1) % K

                @pl.when(tn < NTC)
                def _pref():
                    pltpu.async_copy(src(tn), stgs[q], sems[q])

                pltpu.async_copy(stg, out_hbm.at[dst_idx(t)], wsem)

                @pl.when(j >= 2)
                def _lag():
                    pltpu.make_async_copy(out_hbm.at[0], stg, wsem).wait()

        return carry

    lax.fori_loop(0, TPW, step, 0)
    for _ in range(2):
        pltpu.make_async_copy(out_hbm.at[0], stgs[0], wsem).wait()

    for k in range(-(-NTAIL // NUM_WORKERS)):
        u = wid + k * NUM_WORKERS

        @pl.when(u < NTAIL)
        def _tail(u=u):
            pltpu.sync_copy(tail_hbm.at[u], out_hbm.at[u * VT + VTF])


def _gather_body(cat_hbm, tab_hbm, out_hbm, catv, idxv, dstv, sem):
    wid = lax.axis_index("s") * NUM_CORES + lax.axis_index("c")
    b0 = wid * BPW

    pltpu.sync_copy(cat_hbm.at[:, pl.ds(b0, BPW)], catv)

    for q in range(NQ):
        for f in range(NUM_FIELDS):
            vv0 = catv[f, pl.ds(q * QB, LANES)]
            vv1 = catv[f, pl.ds(q * QB + LANES, LANES)]
            vt0 = ((vv0 >> 7) << 10) + (vv0 & 127)
            vt1 = ((vv1 >> 7) << 10) + (vv1 & 127)

            def build(d, carry, vt0=vt0, vt1=vt1, f=f):
                base = (f * 4 + (d >> 3)) * FROW + (d & 7) * CHUNK
                j0 = (f * EMBED_DIM + d) * QB
                row = j0 >> 7
                col = j0 & (CHUNK - 1)
                idxv[row, pl.ds(col, LANES)] = vt0 + base
                idxv[row, pl.ds(col + LANES, LANES)] = vt1 + base
                return carry

            lax.fori_loop(0, EMBED_DIM, build, 0)

        def fire(c, carry):
            pltpu.async_copy(tab_hbm.at[idxv.at[c]], dstv.at[c], sem)
            return carry

        lax.fori_loop(0, NCH, fire, 0)
        pltpu.make_async_copy(out_hbm.at[wid, q], dstv, sem).wait()

        pltpu.sync_copy(dstv, out_hbm.at[wid, q])


@jax.jit
def _cat_embedding(cat_features, tables_t, tail_tiles):
    mesh = plsc.VectorSubcoreMesh(core_axis_name="c", subcore_axis_name="s")

    copy_run = pl.kernel(
        _copy_body,
        out_type=jax.ShapeDtypeStruct((NTILES, 8, 128), jnp.float32),
        mesh=mesh,
        scratch_types=(
            [pltpu.VMEM((8, 128), jnp.float32)] * 12
            + [pltpu.SemaphoreType.DMA] * 13
        ),
        compiler_params=pltpu.CompilerParams(
            needs_layout_passes=False, use_tc_tiling_on_sc=True
        ),
    )
    flat = copy_run(tables_t, tail_tiles).reshape(NTILES * 1024)

    gather_run = pl.kernel(
        _gather_body,
        out_type=jax.ShapeDtypeStruct(
            (NUM_WORKERS, NQ, NCH, CHUNK), jnp.float32
        ),
        mesh=mesh,
        scratch_types=[
            pltpu.VMEM((NUM_FIELDS, BPW), jnp.int32),
            pltpu.VMEM((NCH, CHUNK), jnp.int32),
            pltpu.VMEM((NCH, CHUNK), jnp.float32),
            pltpu.SemaphoreType.DMA,
        ],
        compiler_params=pltpu.CompilerParams(
            needs_layout_passes=False, use_tc_tiling_on_sc=False
        ),
    )
    return gather_run(cat_features, flat)


def kernel(cat_features, tables):
    cat = cat_features.astype(jnp.int32)
    tab_t = jnp.transpose(tables, (0, 2, 1))
    tail = jnp.transpose(tables[:, 99968:, :], (0, 2, 1))
    tail = jnp.pad(tail, ((0, 0), (0, 0), (0, 95)))
    out = _cat_embedding(cat, tab_t, tail.reshape(NTAIL, 8, 128))
    out = out.reshape(NUM_WORKERS, NQ, NUM_FIELDS, EMBED_DIM, QB)
    out = jnp.transpose(out, (0, 1, 4, 2, 3))
    return out.reshape(BATCH, NUM_FIELDS * EMBED_DIM)
